# baseline (speedup 1.0000x reference)
import functools

import jax
import jax.numpy as jnp
from jax import lax
from jax.experimental import pallas as pl
from jax.experimental.pallas import tpu as pltpu
from jax.experimental.pallas import tpu_sc as plsc

B, C, H, W = 2, 128, 60, 80
HW = H * W
NVOX = 60 * 36 * 60
NTILES = 32
ROWS1 = 4808
TROWS = 2 * ROWS1
VCH = 64
NR = 2 * VCH
NCHUNK = NVOX // VCH
NDEPTH = 3
NITER = 66
GSPLITS = (0,)


def _sc_body(feat, idx, out, table_s, zbuf, idxb, cidx, rows_v,
             gsem, isem0, isem1, isem2, osem0, osem1, osem2):
    cid = lax.axis_index("c")
    sid = lax.axis_index("s")
    wid = sid * 2 + cid

    zeros16 = jnp.zeros((16,), jnp.float32)
    for i in range(8):
        for j in range(8):
            zbuf[i, pl.ds(j * 16, 16)] = zeros16
    seg0 = pl.multiple_of((sid % 8) * 600, 8)

    @pl.when(sid < 8)
    def _():
        for b in range(2):
            pltpu.sync_copy(feat.at[pl.ds(b * HW + seg0, 304), :],
                            table_s.at[pl.ds(b * ROWS1 + seg0, 304), :])

    @pl.when(sid >= 8)
    def _():
        for b in range(2):
            pltpu.sync_copy(
                feat.at[pl.ds(b * HW + seg0 + 304, 296), :],
                table_s.at[pl.ds(b * ROWS1 + seg0 + 304, 296), :])
    @pl.when(sid == 0)
    def _():
        pltpu.sync_copy(zbuf, table_s.at[pl.ds(HW, 8), :])
        pltpu.sync_copy(zbuf, table_s.at[pl.ds(ROWS1 + HW, 8), :])
    plsc.subcore_barrier()

    iota2 = lax.iota(jnp.int32, 16) * 2
    osem = (osem0, osem1, osem2)
    isem = (isem0, isem1, isem2)

    def fire_idx(ck, slot):
        v0 = ck * VCH
        rbase = slot * NR
        pltpu.async_copy(idx.at[pl.ds(v0, VCH)],
                         idxb.at[pl.ds(rbase, VCH)], isem[slot])
        pltpu.async_copy(idx.at[pl.ds(NVOX + v0, VCH)],
                         idxb.at[pl.ds(rbase + VCH, VCH)], isem[slot])

    def wait_idx(slot):
        rbase = slot * NR
        pltpu.make_async_copy(idx.at[pl.ds(0, VCH)],
                              idxb.at[pl.ds(rbase, VCH)], isem[slot]).wait()
        pltpu.make_async_copy(idx.at[pl.ds(0, VCH)],
                              idxb.at[pl.ds(rbase + VCH, VCH)],
                              isem[slot]).wait()

    fire_idx(wid, 0)

    @pl.loop(0, NITER, step=NDEPTH)
    def round_pair(t0):
        for p in range(NDEPTH):
            t = t0 + p
            ck = wid + NTILES * t

            @pl.when(ck < NCHUNK)
            def _(p=p, t=t, ck=ck):
                v0 = ck * VCH
                rbase = p * NR

                @pl.when(ck + NTILES < NCHUNK)
                def _():
                    fire_idx(ck + NTILES, (p + 1) % NDEPTH)

                @pl.when(t >= NDEPTH)
                def _():
                    pltpu.make_async_copy(rows_v.at[pl.ds(rbase, NR), :],
                                          out.at[pl.ds(0, NR), :],
                                          osem[p]).wait()
                wait_idx(p)

                for g in range(VCH // 16):
                    pos = iota2 + g * 32
                    iv0 = idxb[pl.ds(rbase + g * 16, 16)]
                    iv1 = idxb[pl.ds(rbase + VCH + g * 16, 16)] + ROWS1
                    plsc.store_scatter(cidx, [pos], iv0)
                    plsc.store_scatter(cidx, [pos + 1], iv1)

                descs = [pltpu.async_copy(
                    table_s.at[cidx.at[pl.ds(g0, min(128, NR - g0))]],
                    rows_v.at[pl.ds(rbase + g0, min(128, NR - g0)), :],
                    gsem) for g0 in GSPLITS]
                for d in descs:
                    d.wait()
                pltpu.async_copy(rows_v.at[pl.ds(rbase, NR), :],
                                 out.at[pl.ds(v0 * 2, NR), :], osem[p])

    for p in range(NDEPTH):
        pltpu.make_async_copy(rows_v.at[pl.ds(p * NR, NR), :],
                              out.at[pl.ds(0, NR), :], osem[p]).wait()


_sc_call = pl.kernel(
    _sc_body,
    mesh=plsc.VectorSubcoreMesh(core_axis_name="c", subcore_axis_name="s"),
    compiler_params=pltpu.CompilerParams(needs_layout_passes=False),
    out_type=jax.ShapeDtypeStruct((2 * NVOX, C), jnp.float32),
    scratch_types=[
        pltpu.VMEM_SHARED((TROWS, C), jnp.float32),
        pltpu.VMEM((8, C), jnp.float32),
        pltpu.VMEM((NDEPTH * NR,), jnp.int32),
        pltpu.VMEM((NR,), jnp.int32),
        pltpu.VMEM((NDEPTH * NR, C), jnp.float32),
        pltpu.SemaphoreType.DMA,
        pltpu.SemaphoreType.DMA,
        pltpu.SemaphoreType.DMA,
        pltpu.SemaphoreType.DMA,
        pltpu.SemaphoreType.DMA,
        pltpu.SemaphoreType.DMA,
        pltpu.SemaphoreType.DMA,
    ],
)


@jax.jit
def kernel(feature2d, depth_mapping_3d):
    feat = feature2d.transpose(0, 2, 3, 1).reshape(B * HW, C)
    out = _sc_call(feat, depth_mapping_3d.reshape(B * NVOX))
    out = out.reshape(NVOX, B, C).transpose(1, 2, 0)
    return out.reshape(B, C, 60, 36, 60)


# speedup vs baseline: 1.0125x; 1.0125x over previous
import jax
import jax.numpy as jnp
from jax import lax
from jax.experimental import pallas as pl
from jax.experimental.pallas import tpu as pltpu
from jax.experimental.pallas import tpu_sc as plsc

B, C, H, W = 2, 128, 60, 80
HW = H * W
NVOX = 60 * 36 * 60
NTILES = 32
ROWS1 = 4808
TROWS = 2 * ROWS1
VCH = 96
NR = 2 * VCH
NCHUNK = NVOX // VCH
NITER = 44
GSPLITS = (0, 128)


def _sc_body(feat, idx, out, table_s, zbuf, idxb, cidx, rows_v,
             gsem, isem0, isem1, osem0, osem1):
    cid = lax.axis_index("c")
    sid = lax.axis_index("s")
    wid = sid * 2 + cid

    zeros16 = jnp.zeros((16,), jnp.float32)
    for i in range(8):
        for j in range(8):
            zbuf[i, pl.ds(j * 16, 16)] = zeros16
    seg0 = pl.multiple_of((sid % 8) * 600, 8)

    @pl.when(sid < 8)
    def _():
        for b in range(2):
            pltpu.sync_copy(feat.at[pl.ds(b * HW + seg0, 304), :],
                            table_s.at[pl.ds(b * ROWS1 + seg0, 304), :])

    @pl.when(sid >= 8)
    def _():
        for b in range(2):
            pltpu.sync_copy(
                feat.at[pl.ds(b * HW + seg0 + 304, 296), :],
                table_s.at[pl.ds(b * ROWS1 + seg0 + 304, 296), :])
    @pl.when(sid == 0)
    def _():
        pltpu.sync_copy(zbuf, table_s.at[pl.ds(HW, 8), :])
        pltpu.sync_copy(zbuf, table_s.at[pl.ds(ROWS1 + HW, 8), :])
    plsc.subcore_barrier()

    iota2 = lax.iota(jnp.int32, 16) * 2
    osem = (osem0, osem1)
    isem = (isem0, isem1)

    def fire_idx(ck, slot):
        v0 = ck * VCH
        rbase = slot * NR
        pltpu.async_copy(idx.at[pl.ds(v0, VCH)],
                         idxb.at[pl.ds(rbase, VCH)], isem[slot])
        pltpu.async_copy(idx.at[pl.ds(NVOX + v0, VCH)],
                         idxb.at[pl.ds(rbase + VCH, VCH)], isem[slot])

    def wait_idx(slot):
        rbase = slot * NR
        pltpu.make_async_copy(idx.at[pl.ds(0, VCH)],
                              idxb.at[pl.ds(rbase, VCH)], isem[slot]).wait()
        pltpu.make_async_copy(idx.at[pl.ds(0, VCH)],
                              idxb.at[pl.ds(rbase + VCH, VCH)],
                              isem[slot]).wait()

    fire_idx(wid, 0)

    @pl.loop(0, NITER, step=2)
    def round_pair(t0):
        for p in range(2):
            t = t0 + p
            ck = wid + NTILES * t

            @pl.when(ck < NCHUNK)
            def _(p=p, t=t, ck=ck):
                v0 = ck * VCH
                rbase = p * NR

                @pl.when(ck + NTILES < NCHUNK)
                def _():
                    fire_idx(ck + NTILES, 1 - p)

                @pl.when(t >= 2)
                def _():
                    pltpu.make_async_copy(rows_v.at[pl.ds(rbase, NR), :],
                                          out.at[pl.ds(0, NR), :],
                                          osem[p]).wait()
                wait_idx(p)

                for g in range(VCH // 16):
                    pos = iota2 + g * 32
                    iv0 = idxb[pl.ds(rbase + g * 16, 16)]
                    iv1 = idxb[pl.ds(rbase + VCH + g * 16, 16)] + ROWS1
                    plsc.store_scatter(cidx, [pos], iv0)
                    plsc.store_scatter(cidx, [pos + 1], iv1)

                descs = [pltpu.async_copy(
                    table_s.at[cidx.at[pl.ds(g0, min(128, NR - g0))]],
                    rows_v.at[pl.ds(rbase + g0, min(128, NR - g0)), :],
                    gsem) for g0 in GSPLITS]
                for d in descs:
                    d.wait()
                pltpu.async_copy(rows_v.at[pl.ds(rbase, NR), :],
                                 out.at[pl.ds(v0 * 2, NR), :], osem[p])

    for p in range(2):
        pltpu.make_async_copy(rows_v.at[pl.ds(p * NR, NR), :],
                              out.at[pl.ds(0, NR), :], osem[p]).wait()


_sc_call = pl.kernel(
    _sc_body,
    mesh=plsc.VectorSubcoreMesh(core_axis_name="c", subcore_axis_name="s"),
    compiler_params=pltpu.CompilerParams(needs_layout_passes=False),
    out_type=jax.ShapeDtypeStruct((2 * NVOX, C), jnp.float32),
    scratch_types=[
        pltpu.VMEM_SHARED((TROWS, C), jnp.float32),
        pltpu.VMEM((8, C), jnp.float32),
        pltpu.VMEM((2 * NR,), jnp.int32),
        pltpu.VMEM((NR,), jnp.int32),
        pltpu.VMEM((2 * NR, C), jnp.float32),
        pltpu.SemaphoreType.DMA,
        pltpu.SemaphoreType.DMA,
        pltpu.SemaphoreType.DMA,
        pltpu.SemaphoreType.DMA,
        pltpu.SemaphoreType.DMA,
    ],
)


@jax.jit
def kernel(feature2d, depth_mapping_3d):
    feat = feature2d.transpose(0, 2, 3, 1).reshape(B * HW, C)
    out = _sc_call(feat, depth_mapping_3d.reshape(B * NVOX))
    out = out.reshape(NVOX, B, C).transpose(1, 2, 0)
    return out.reshape(B, C, 60, 36, 60)
